# baseline (speedup 1.0000x reference)
import functools

import jax
import jax.numpy as jnp
from jax.experimental import pallas as pl
from jax.experimental.pallas import tpu as pltpu

TB = 128
CB = 1024


def _gmm_body(bex_ref, xs_ref, w_ref, a_ref, b_ref, dw_ref, metas_ref, ss_ref,
              out_ref):
    j = pl.program_id(0)
    xs32 = xs_ref[...]
    xb = xs32.astype(jnp.bfloat16)
    base = jax.lax.dot_general(
        xb, w_ref[...].astype(jnp.bfloat16), (((1,), (1,)), ((), ())),
        preferred_element_type=jnp.float32)
    mid = jnp.dot(xb, a_ref[0].astype(jnp.bfloat16),
                  preferred_element_type=jnp.float32)
    b = b_ref[0].astype(jnp.bfloat16)
    zr = jnp.zeros_like(b)
    bfull = jnp.where(j == 0,
                      jnp.concatenate([b, zr], axis=0),
                      jnp.concatenate([zr, b], axis=0))
    lora = jnp.dot(mid.astype(jnp.bfloat16), bfull,
                   preferred_element_type=jnp.float32)
    dmm = jnp.dot(xb, dw_ref[0].astype(jnp.bfloat16),
                  preferred_element_type=jnp.float32)
    rs = jnp.sum(xs32, axis=1, keepdims=True)
    out_ref[...] = base + lora + dmm * ss_ref[0] + rs * metas_ref[0]


def _grouped_matmul(bex, xs, W, A, B, DW, metas, ss, P):
    E, D, R2 = A.shape
    R = R2 // 2
    OUT = B.shape[2]
    nj = OUT // CB
    ntb = P // TB
    grid = (nj, ntb)

    def tok(j, tb, bex_ref):
        return (tb, 0)

    def wmap(j, tb, bex_ref):
        return (j, 0)

    def emap(j, tb, bex_ref):
        return (bex_ref[tb], 0, j)

    grid_spec = pltpu.PrefetchScalarGridSpec(
        num_scalar_prefetch=1,
        grid=grid,
        in_specs=[
            pl.BlockSpec((TB, D), tok),
            pl.BlockSpec((CB, D), wmap),
            pl.BlockSpec((1, D, R2), lambda j, tb, bex_ref: (bex_ref[tb], 0, 0)),
            pl.BlockSpec((1, R, CB), emap),
            pl.BlockSpec((1, D, CB), emap),
            pl.BlockSpec((1, 1, CB), emap),
            pl.BlockSpec((1, 1, CB), emap),
        ],
        out_specs=pl.BlockSpec((TB, CB), lambda j, tb, bex_ref: (tb, j)),
    )
    return pl.pallas_call(
        _gmm_body,
        grid_spec=grid_spec,
        out_shape=jax.ShapeDtypeStruct((P, OUT), jnp.float32),
        compiler_params=pltpu.CompilerParams(
            dimension_semantics=("arbitrary", "arbitrary")),
    )(bex, xs, W, A, B, DW, metas, ss)


def kernel(input_, weight_indices, W, A_buffer, B_buffer, DeltaW_buffer,
           metas_buffer, ss_buffer):
    T, D = input_.shape
    E = A_buffer.shape[0]
    P = T + E * TB
    idx = weight_indices.astype(jnp.int32)

    onehot = jax.nn.one_hot(idx, E, dtype=jnp.int32)
    counts = jnp.sum(onehot, axis=0)
    seg = ((counts + TB - 1) // TB) * TB
    segend = jnp.cumsum(seg)
    off = segend - seg
    occ = jnp.cumsum(onehot, axis=0) - onehot
    rank = jnp.take_along_axis(occ, idx[:, None], axis=1)[:, 0]
    pos = jnp.take(off, idx) + rank
    perm = jnp.zeros((P,), jnp.int32).at[pos].set(jnp.arange(T, dtype=jnp.int32))
    bex = jnp.clip(
        jnp.searchsorted(segend, jnp.arange(P // TB, dtype=jnp.int32) * TB,
                         side="right"), 0, E - 1).astype(jnp.int32)

    xs = jnp.take(input_, perm, axis=0)

    out_sorted = _grouped_matmul(bex, xs, W, A_buffer, B_buffer, DeltaW_buffer,
                                 metas_buffer, ss_buffer, P)

    return jnp.take(out_sorted, pos, axis=0)


# speedup vs baseline: 1.5010x; 1.5010x over previous
import functools

import jax
import jax.numpy as jnp
from jax import lax
from jax.experimental import pallas as pl
from jax.experimental.pallas import tpu as pltpu
from jax.experimental.pallas import tpu_sc as plsc

TB = 128
CB = 1024
L = 16


def _make_route_gather(T, D, E, P):
    NC, NS = 2, 16
    TPW = T // NS
    HPW = TPW // 2
    mesh = plsc.VectorSubcoreMesh(core_axis_name="c", subcore_axis_name="s")

    @functools.partial(
        pl.kernel, mesh=mesh,
        out_type=[
            jax.ShapeDtypeStruct((P, D), jnp.float32),
            jax.ShapeDtypeStruct((T,), jnp.int32),
            jax.ShapeDtypeStruct((2 * L,), jnp.int32),
        ],
        scratch_types=[
            pltpu.VMEM((TPW,), jnp.int32),
            pltpu.VMEM((HPW,), jnp.int32),
            pltpu.VMEM((HPW,), jnp.int32),
            pltpu.VMEM((HPW, D), jnp.float32),
            pltpu.VMEM((L,), jnp.int32),
            pltpu.VMEM((NS * L,), jnp.int32),
            pltpu.VMEM((2 * L,), jnp.int32),
            pltpu.VMEM_SHARED((NS * L,), jnp.int32),
            pltpu.SemaphoreType.DMA,
        ],
    )
    def route_gather(idx_hbm, x_hbm, xs_hbm, pos_hbm, bex_hbm,
                     idx_v, posva, posvb, rows, cntbuf, cmatv, bexbuf,
                     cnt_sh, sem):
        c = lax.axis_index("c")
        s = lax.axis_index("s")
        lanes = lax.iota(jnp.int32, L)
        sbase = pl.multiple_of(s * TPW, TPW)
        pltpu.sync_copy(idx_hbm.at[pl.ds(sbase, TPW)], idx_v)

        def lane_gather(v, idx):
            return lax.gather(
                v, idx[:, None],
                dimension_numbers=lax.GatherDimensionNumbers(
                    offset_dims=(), collapsed_slice_dims=(0,),
                    start_index_map=(0,)),
                slice_sizes=(1,),
                mode=lax.GatherScatterMode.PROMISE_IN_BOUNDS)

        def splat(v, k):
            return lane_gather(v, jnp.full((L,), k, jnp.int32))

        def prefix_incl(v):
            for k in (1, 2, 4, 8):
                shifted = lane_gather(v, jnp.maximum(lanes - k, 0))
                v = v + jnp.where(lanes >= k, shifted, 0)
            return v

        def chunk_ranks(v, rstart):
            vals = jnp.zeros((L,), jnp.int32)
            add = jnp.zeros((L,), jnp.int32)
            for e in range(E):
                m = v == e
                mi = jnp.where(m, jnp.int32(1), jnp.int32(0))
                pres = prefix_incl(mi)
                tot = splat(pres, L - 1)
                vals = jnp.where(m, splat(rstart, e) + pres - 1, vals)
                add = add + jnp.where(lanes == e, tot, 0)
            return vals, rstart + add

        def cnt_body(i, rs):
            v = idx_v[pl.ds(i * L, L)]
            _, rs = chunk_ranks(v, rs)
            return rs

        mycnt = lax.fori_loop(0, TPW // L, cnt_body,
                              jnp.zeros((L,), jnp.int32))
        cntbuf[...] = mycnt
        pltpu.sync_copy(cntbuf,
                        cnt_sh.at[pl.ds(pl.multiple_of(s * L, L), L)])
        plsc.subcore_barrier()
        pltpu.sync_copy(cnt_sh, cmatv)

        total = jnp.zeros((L,), jnp.int32)
        mypre = jnp.zeros((L,), jnp.int32)
        for s2 in range(NS):
            row = cmatv[pl.ds(s2 * L, L)]
            total = total + row
            mypre = mypre + jnp.where(s2 < s, row, 0)
        seg = jnp.bitwise_and(total + (TB - 1), -TB)
        segend = prefix_incl(seg)
        off = segend - seg
        rstart0 = off + mypre

        def pos_body(tref, base_ci):
            def body(i, rs):
                v = idx_v[pl.ds(pl.multiple_of((base_ci + i) * L, L), L)]
                vals, rs = chunk_ranks(v, rs)
                tref[pl.ds(i * L, L)] = vals
                return rs

            return body

        rs1 = lax.fori_loop(0, HPW // L, pos_body(posva, 0), rstart0)
        lax.fori_loop(0, HPW // L, pos_body(posvb, HPW // L), rs1)

        @pl.when(c == 0)
        def _wpos():
            pltpu.sync_copy(posva, pos_hbm.at[pl.ds(sbase, HPW)])
            pltpu.sync_copy(posvb, pos_hbm.at[pl.ds(sbase + HPW, HPW)])

        @pl.when((c == 0) & (s == 0))
        def _bex():
            b1 = jnp.zeros((L,), jnp.int32)
            b2 = jnp.zeros((L,), jnp.int32)
            for e in range(E):
                sev = splat(segend, e)
                b1 = b1 + jnp.where(lanes * TB >= sev, 1, 0).astype(jnp.int32)
                b2 = b2 + jnp.where((lanes + L) * TB >= sev,
                                    1, 0).astype(jnp.int32)
            bexbuf[pl.ds(0, L)] = jnp.minimum(b1, E - 1)
            bexbuf[pl.ds(L, L)] = jnp.minimum(b2, E - 1)
            pltpu.sync_copy(bexbuf, bex_hbm)

        @pl.when(c == 0)
        def _sc0():
            pltpu.sync_copy(x_hbm.at[pl.ds(sbase, HPW)], rows)
            pltpu.async_copy(rows, xs_hbm.at[posva], sem).wait()

        @pl.when(c == 1)
        def _sc1():
            pltpu.sync_copy(x_hbm.at[pl.ds(sbase + HPW, HPW)], rows)
            pltpu.async_copy(rows, xs_hbm.at[posvb], sem).wait()

    return route_gather


def _make_out_gather(T, OUT, P):
    NC, NS = 2, 16
    NW = NC * NS
    RPW = T // NW
    HC = RPW // 4
    mesh = plsc.VectorSubcoreMesh(core_axis_name="c", subcore_axis_name="s")

    @functools.partial(
        pl.kernel, mesh=mesh,
        out_type=jax.ShapeDtypeStruct((T, OUT), jnp.float32),
        scratch_types=[
            pltpu.VMEM((HC,), jnp.int32),
            pltpu.VMEM((HC,), jnp.int32),
            pltpu.VMEM((HC, OUT), jnp.float32),
            pltpu.VMEM((HC, OUT), jnp.float32),
            pltpu.SemaphoreType.DMA,
            pltpu.SemaphoreType.DMA,
        ],
    )
    def out_gather(outs_hbm, pos_hbm, out_hbm, posa, posb, rowsa, rowsb,
                   sema, semb):
        c = lax.axis_index("c")
        s = lax.axis_index("s")
        wid = s * NC + c
        base = pl.multiple_of(RPW * wid, RPW)
        pltpu.sync_copy(pos_hbm.at[pl.ds(base, HC)], posa)
        cps = [None, None]
        bufs = [rowsa, rowsb]
        sems = [sema, semb]
        cps[0] = pltpu.async_copy(outs_hbm.at[posa], rowsa, sema)
        for h in range(1, 4):
            pbuf = posa if (h % 2 == 0) else posb
            pltpu.sync_copy(pos_hbm.at[pl.ds(base + h * HC, HC)], pbuf)
            cps[h % 2] = pltpu.async_copy(outs_hbm.at[pbuf], bufs[h % 2],
                                          sems[h % 2])
            cps[(h - 1) % 2].wait()
            pltpu.sync_copy(bufs[(h - 1) % 2],
                            out_hbm.at[pl.ds(base + (h - 1) * HC, HC)])
        cps[1].wait()
        pltpu.sync_copy(bufs[1], out_hbm.at[pl.ds(base + 3 * HC, HC)])

    return out_gather


def _gmm_body(bex_ref, xs_ref, w_ref, a_ref, b_ref, dw_ref, metas_ref, ss_ref,
              out_ref):
    j = pl.program_id(0)
    xs32 = xs_ref[...]
    xb = xs32.astype(jnp.bfloat16)
    base = jax.lax.dot_general(
        xb, w_ref[...], (((1,), (1,)), ((), ())),
        preferred_element_type=jnp.float32)
    mid = jnp.dot(xb, a_ref[0], preferred_element_type=jnp.float32)
    b = b_ref[0]
    zr = jnp.zeros_like(b)
    bfull = jnp.where(j == 0,
                      jnp.concatenate([b, zr], axis=0),
                      jnp.concatenate([zr, b], axis=0))
    lora = jnp.dot(mid.astype(jnp.bfloat16), bfull,
                   preferred_element_type=jnp.float32)
    dmm = jnp.dot(xb, dw_ref[0], preferred_element_type=jnp.float32)
    rs = jnp.sum(xs32, axis=1, keepdims=True)
    out_ref[...] = base + lora + dmm * ss_ref[0] + rs * metas_ref[0]


def _grouped_matmul(bex, xs, W, A, B, DW, metas, ss, P):
    E, D, R2 = A.shape
    R = R2 // 2
    OUT = B.shape[2]
    nj = OUT // CB
    ntb = P // TB
    grid = (nj, ntb)

    def tok(j, tb, bex_ref):
        return (tb, 0)

    def wmap(j, tb, bex_ref):
        return (j, 0)

    def emap(j, tb, bex_ref):
        return (bex_ref[tb], 0, j)

    grid_spec = pltpu.PrefetchScalarGridSpec(
        num_scalar_prefetch=1,
        grid=grid,
        in_specs=[
            pl.BlockSpec((TB, D), tok),
            pl.BlockSpec((CB, D), wmap),
            pl.BlockSpec((1, D, R2), lambda j, tb, bex_ref: (bex_ref[tb], 0, 0)),
            pl.BlockSpec((1, R, CB), emap),
            pl.BlockSpec((1, D, CB), emap),
            pl.BlockSpec((1, 1, CB), emap),
            pl.BlockSpec((1, 1, CB), emap),
        ],
        out_specs=pl.BlockSpec((TB, CB), lambda j, tb, bex_ref: (tb, j)),
    )
    return pl.pallas_call(
        _gmm_body,
        grid_spec=grid_spec,
        out_shape=jax.ShapeDtypeStruct((P, OUT), jnp.float32),
        compiler_params=pltpu.CompilerParams(
            dimension_semantics=("arbitrary", "arbitrary")),
    )(bex, xs, W, A, B, DW, metas, ss)


def kernel(input_, weight_indices, W, A_buffer, B_buffer, DeltaW_buffer,
           metas_buffer, ss_buffer):
    T, D = input_.shape
    E = A_buffer.shape[0]
    OUT = B_buffer.shape[2]
    P = T + E * TB
    idx = weight_indices.astype(jnp.int32)

    xs, pos, bex = _make_route_gather(T, D, E, P)(idx, input_)

    out_sorted = _grouped_matmul(
        bex, xs, W.astype(jnp.bfloat16), A_buffer.astype(jnp.bfloat16),
        B_buffer.astype(jnp.bfloat16), DeltaW_buffer.astype(jnp.bfloat16),
        metas_buffer, ss_buffer, P)

    return _make_out_gather(T, OUT, P)(out_sorted, pos)


# speedup vs baseline: 1.6705x; 1.1129x over previous
import functools

import jax
import jax.numpy as jnp
from jax import lax
from jax.experimental import pallas as pl
from jax.experimental.pallas import tpu as pltpu
from jax.experimental.pallas import tpu_sc as plsc

TB = 128
CB = 1024
L = 16


def _make_route_gather(T, D, E, P):
    NC, NS = 2, 16
    TPW = T // NS
    HPW = TPW // 2
    mesh = plsc.VectorSubcoreMesh(core_axis_name="c", subcore_axis_name="s")

    @functools.partial(
        pl.kernel, mesh=mesh,
        out_type=[
            jax.ShapeDtypeStruct((P, D), jnp.float32),
            jax.ShapeDtypeStruct((T,), jnp.int32),
            jax.ShapeDtypeStruct((2 * L,), jnp.int32),
        ],
        scratch_types=[
            pltpu.VMEM((TPW,), jnp.int32),
            pltpu.VMEM((HPW,), jnp.int32),
            pltpu.VMEM((HPW,), jnp.int32),
            pltpu.VMEM((HPW, D), jnp.float32),
            pltpu.VMEM((L,), jnp.int32),
            pltpu.VMEM((NS * L,), jnp.int32),
            pltpu.VMEM((2 * L,), jnp.int32),
            pltpu.VMEM_SHARED((NS * L,), jnp.int32),
            pltpu.SemaphoreType.DMA,
        ],
    )
    def route_gather(idx_hbm, x_hbm, xs_hbm, pos_hbm, bex_hbm,
                     idx_v, posva, posvb, rows, cntbuf, cmatv, bexbuf,
                     cnt_sh, sem):
        c = lax.axis_index("c")
        s = lax.axis_index("s")
        lanes = lax.iota(jnp.int32, L)
        sbase = pl.multiple_of(s * TPW, TPW)
        pltpu.sync_copy(idx_hbm.at[pl.ds(sbase, TPW)], idx_v)

        def lane_gather(v, idx):
            return lax.gather(
                v, idx[:, None],
                dimension_numbers=lax.GatherDimensionNumbers(
                    offset_dims=(), collapsed_slice_dims=(0,),
                    start_index_map=(0,)),
                slice_sizes=(1,),
                mode=lax.GatherScatterMode.PROMISE_IN_BOUNDS)

        def splat(v, k):
            return lane_gather(v, jnp.full((L,), k, jnp.int32))

        def prefix_incl(v):
            for k in (1, 2, 4, 8):
                shifted = lane_gather(v, jnp.maximum(lanes - k, 0))
                v = v + jnp.where(lanes >= k, shifted, 0)
            return v

        def chunk_ranks(v, rstart):
            vals = jnp.zeros((L,), jnp.int32)
            add = jnp.zeros((L,), jnp.int32)
            for e in range(E):
                m = v == e
                mi = jnp.where(m, jnp.int32(1), jnp.int32(0))
                pres = prefix_incl(mi)
                tot = splat(pres, L - 1)
                vals = jnp.where(m, splat(rstart, e) + pres - 1, vals)
                add = add + jnp.where(lanes == e, tot, 0)
            return vals, rstart + add

        def cnt_body(i, rs):
            v = idx_v[pl.ds(i * L, L)]
            _, rs = chunk_ranks(v, rs)
            return rs

        mycnt = lax.fori_loop(0, TPW // L, cnt_body,
                              jnp.zeros((L,), jnp.int32))
        cntbuf[...] = mycnt
        pltpu.sync_copy(cntbuf,
                        cnt_sh.at[pl.ds(pl.multiple_of(s * L, L), L)])
        plsc.subcore_barrier()
        pltpu.sync_copy(cnt_sh, cmatv)

        total = jnp.zeros((L,), jnp.int32)
        mypre = jnp.zeros((L,), jnp.int32)
        for s2 in range(NS):
            row = cmatv[pl.ds(s2 * L, L)]
            total = total + row
            mypre = mypre + jnp.where(s2 < s, row, 0)
        seg = jnp.bitwise_and(total + (TB - 1), -TB)
        segend = prefix_incl(seg)
        off = segend - seg
        rstart0 = off + mypre

        def pos_body(tref, base_ci):
            def body(i, rs):
                v = idx_v[pl.ds(pl.multiple_of((base_ci + i) * L, L), L)]
                vals, rs = chunk_ranks(v, rs)
                tref[pl.ds(i * L, L)] = vals
                return rs

            return body

        rs1 = lax.fori_loop(0, HPW // L, pos_body(posva, 0), rstart0)
        lax.fori_loop(0, HPW // L, pos_body(posvb, HPW // L), rs1)

        @pl.when(c == 0)
        def _wpos():
            pltpu.sync_copy(posva, pos_hbm.at[pl.ds(sbase, HPW)])
            pltpu.sync_copy(posvb, pos_hbm.at[pl.ds(sbase + HPW, HPW)])

        @pl.when((c == 0) & (s == 0))
        def _bex():
            b1 = jnp.zeros((L,), jnp.int32)
            b2 = jnp.zeros((L,), jnp.int32)
            for e in range(E):
                sev = splat(segend, e)
                b1 = b1 + jnp.where(lanes * TB >= sev, 1, 0).astype(jnp.int32)
                b2 = b2 + jnp.where((lanes + L) * TB >= sev,
                                    1, 0).astype(jnp.int32)
            bexbuf[pl.ds(0, L)] = jnp.minimum(b1, E - 1)
            bexbuf[pl.ds(L, L)] = jnp.minimum(b2, E - 1)
            pltpu.sync_copy(bexbuf, bex_hbm)

        @pl.when(c == 0)
        def _sc0():
            pltpu.sync_copy(x_hbm.at[pl.ds(sbase, HPW)], rows)
            pltpu.async_copy(rows, xs_hbm.at[posva], sem).wait()

        @pl.when(c == 1)
        def _sc1():
            pltpu.sync_copy(x_hbm.at[pl.ds(sbase + HPW, HPW)], rows)
            pltpu.async_copy(rows, xs_hbm.at[posvb], sem).wait()

    return route_gather


def _make_out_gather(T, OUT, P):
    NC, NS = 2, 16
    NW = NC * NS
    RPW = T // NW
    HC = RPW // 4
    mesh = plsc.VectorSubcoreMesh(core_axis_name="c", subcore_axis_name="s")

    @functools.partial(
        pl.kernel, mesh=mesh,
        out_type=jax.ShapeDtypeStruct((T, OUT), jnp.float32),
        scratch_types=[
            pltpu.VMEM((HC,), jnp.int32),
            pltpu.VMEM((HC,), jnp.int32),
            pltpu.VMEM((HC, OUT), jnp.float32),
            pltpu.VMEM((HC, OUT), jnp.float32),
            pltpu.SemaphoreType.DMA,
            pltpu.SemaphoreType.DMA,
        ],
    )
    def out_gather(outs_hbm, pos_hbm, out_hbm, posa, posb, rowsa, rowsb,
                   sema, semb):
        c = lax.axis_index("c")
        s = lax.axis_index("s")
        wid = s * NC + c
        base = pl.multiple_of(RPW * wid, RPW)
        pltpu.sync_copy(pos_hbm.at[pl.ds(base, HC)], posa)
        cps = [None, None]
        bufs = [rowsa, rowsb]
        sems = [sema, semb]
        cps[0] = pltpu.async_copy(outs_hbm.at[posa], rowsa, sema)
        for h in range(1, 4):
            pbuf = posa if (h % 2 == 0) else posb
            pltpu.sync_copy(pos_hbm.at[pl.ds(base + h * HC, HC)], pbuf)
            cps[h % 2] = pltpu.async_copy(outs_hbm.at[pbuf], bufs[h % 2],
                                          sems[h % 2])
            cps[(h - 1) % 2].wait()
            pltpu.sync_copy(bufs[(h - 1) % 2],
                            out_hbm.at[pl.ds(base + (h - 1) * HC, HC)])
        cps[1].wait()
        pltpu.sync_copy(bufs[1], out_hbm.at[pl.ds(base + 3 * HC, HC)])

    return out_gather


def _gmm_body(bex_ref, xs_ref, w_ref, a_ref, b_ref, dw_ref, metas_ref, ss_ref,
              out_ref):
    j = pl.program_id(0)
    xs32 = xs_ref[...]
    xb = xs32.astype(jnp.bfloat16)
    base = jax.lax.dot_general(
        xb, w_ref[...].astype(jnp.bfloat16), (((1,), (1,)), ((), ())),
        preferred_element_type=jnp.float32)
    mid = jnp.dot(xb, a_ref[0].astype(jnp.bfloat16),
                  preferred_element_type=jnp.float32)
    b = b_ref[0].astype(jnp.bfloat16)
    zr = jnp.zeros_like(b)
    bfull = jnp.where(j == 0,
                      jnp.concatenate([b, zr], axis=0),
                      jnp.concatenate([zr, b], axis=0))
    lora = jnp.dot(mid.astype(jnp.bfloat16), bfull,
                   preferred_element_type=jnp.float32)
    dmm = jnp.dot(xb, dw_ref[0].astype(jnp.bfloat16),
                  preferred_element_type=jnp.float32)
    rs = jnp.sum(xs32, axis=1, keepdims=True)
    out_ref[...] = base + lora + dmm * ss_ref[0] + rs * metas_ref[0]


def _grouped_matmul(bex, xs, W, A, B, DW, metas, ss, P):
    E, D, R2 = A.shape
    R = R2 // 2
    OUT = B.shape[2]
    nj = OUT // CB
    ntb = P // TB
    grid = (nj, ntb)

    def tok(j, tb, bex_ref):
        return (tb, 0)

    def wmap(j, tb, bex_ref):
        return (j, 0)

    def emap(j, tb, bex_ref):
        return (bex_ref[tb], 0, j)

    grid_spec = pltpu.PrefetchScalarGridSpec(
        num_scalar_prefetch=1,
        grid=grid,
        in_specs=[
            pl.BlockSpec((TB, D), tok),
            pl.BlockSpec((CB, D), wmap),
            pl.BlockSpec((1, D, R2), lambda j, tb, bex_ref: (bex_ref[tb], 0, 0)),
            pl.BlockSpec((1, R, CB), emap),
            pl.BlockSpec((1, D, CB), emap),
            pl.BlockSpec((1, 1, CB), emap),
            pl.BlockSpec((1, 1, CB), emap),
        ],
        out_specs=pl.BlockSpec((TB, CB), lambda j, tb, bex_ref: (tb, j)),
    )
    return pl.pallas_call(
        _gmm_body,
        grid_spec=grid_spec,
        out_shape=jax.ShapeDtypeStruct((P, OUT), jnp.float32),
        compiler_params=pltpu.CompilerParams(
            dimension_semantics=("arbitrary", "arbitrary")),
    )(bex, xs, W, A, B, DW, metas, ss)


def kernel(input_, weight_indices, W, A_buffer, B_buffer, DeltaW_buffer,
           metas_buffer, ss_buffer):
    T, D = input_.shape
    E = A_buffer.shape[0]
    OUT = B_buffer.shape[2]
    P = T + E * TB
    idx = weight_indices.astype(jnp.int32)

    xs, pos, bex = _make_route_gather(T, D, E, P)(idx, input_)

    out_sorted = _grouped_matmul(
        bex, xs, W, A_buffer, B_buffer, DeltaW_buffer,
        metas_buffer, ss_buffer, P)

    return _make_out_gather(T, OUT, P)(out_sorted, pos)


# speedup vs baseline: 1.9085x; 1.1425x over previous
import functools

import jax
import jax.numpy as jnp
from jax import lax
from jax.experimental import pallas as pl
from jax.experimental.pallas import tpu as pltpu
from jax.experimental.pallas import tpu_sc as plsc

TB = 128
CB = 1024
L = 16


def _make_route_gather(T, D, E, P):
    NC, NS = 2, 16
    TPW = T // NS
    HPW = TPW // 2
    mesh = plsc.VectorSubcoreMesh(core_axis_name="c", subcore_axis_name="s")

    @functools.partial(
        pl.kernel, mesh=mesh,
        out_type=[
            jax.ShapeDtypeStruct((P, D), jnp.float32),
            jax.ShapeDtypeStruct((T,), jnp.int32),
            jax.ShapeDtypeStruct((2 * L,), jnp.int32),
        ],
        scratch_types=[
            pltpu.VMEM((TPW,), jnp.int32),
            pltpu.VMEM((HPW,), jnp.int32),
            pltpu.VMEM((HPW,), jnp.int32),
            pltpu.VMEM((HPW, D), jnp.float32),
            pltpu.VMEM((L,), jnp.int32),
            pltpu.VMEM((NS * L,), jnp.int32),
            pltpu.VMEM((2 * L,), jnp.int32),
            pltpu.VMEM_SHARED((NS * L,), jnp.int32),
            pltpu.SemaphoreType.DMA,
        ],
    )
    def route_gather(idx_hbm, x_hbm, xs_hbm, pos_hbm, bex_hbm,
                     idx_v, posva, posvb, rows, cntbuf, cmatv, bexbuf,
                     cnt_sh, sem):
        c = lax.axis_index("c")
        s = lax.axis_index("s")
        lanes = lax.iota(jnp.int32, L)
        sbase = pl.multiple_of(s * TPW, TPW)
        pltpu.sync_copy(idx_hbm.at[pl.ds(sbase, TPW)], idx_v)

        def lane_gather(v, idx):
            return lax.gather(
                v, idx[:, None],
                dimension_numbers=lax.GatherDimensionNumbers(
                    offset_dims=(), collapsed_slice_dims=(0,),
                    start_index_map=(0,)),
                slice_sizes=(1,),
                mode=lax.GatherScatterMode.PROMISE_IN_BOUNDS)

        def splat(v, k):
            return lane_gather(v, jnp.full((L,), k, jnp.int32))

        def prefix_incl(v):
            for k in (1, 2, 4, 8):
                shifted = lane_gather(v, jnp.maximum(lanes - k, 0))
                v = v + jnp.where(lanes >= k, shifted, 0)
            return v

        def chunk_ranks(v, rstart):
            vals = jnp.zeros((L,), jnp.int32)
            add = jnp.zeros((L,), jnp.int32)
            for e in range(E):
                m = v == e
                mi = jnp.where(m, jnp.int32(1), jnp.int32(0))
                pres = prefix_incl(mi)
                tot = splat(pres, L - 1)
                vals = jnp.where(m, splat(rstart, e) + pres - 1, vals)
                add = add + jnp.where(lanes == e, tot, 0)
            return vals, rstart + add

        def cnt_body(i, rs):
            v = idx_v[pl.ds(i * L, L)]
            _, rs = chunk_ranks(v, rs)
            return rs

        mycnt = lax.fori_loop(0, TPW // L, cnt_body,
                              jnp.zeros((L,), jnp.int32))
        cntbuf[...] = mycnt
        pltpu.sync_copy(cntbuf,
                        cnt_sh.at[pl.ds(pl.multiple_of(s * L, L), L)])
        plsc.subcore_barrier()
        pltpu.sync_copy(cnt_sh, cmatv)

        total = jnp.zeros((L,), jnp.int32)
        mypre = jnp.zeros((L,), jnp.int32)
        for s2 in range(NS):
            row = cmatv[pl.ds(s2 * L, L)]
            total = total + row
            mypre = mypre + jnp.where(s2 < s, row, 0)
        seg = jnp.bitwise_and(total + (TB - 1), -TB)
        segend = prefix_incl(seg)
        off = segend - seg
        rstart0 = off + mypre

        def pos_body(tref, base_ci):
            def body(i, rs):
                v = idx_v[pl.ds(pl.multiple_of((base_ci + i) * L, L), L)]
                vals, rs = chunk_ranks(v, rs)
                tref[pl.ds(i * L, L)] = vals
                return rs

            return body

        rs1 = lax.fori_loop(0, HPW // L, pos_body(posva, 0), rstart0)
        lax.fori_loop(0, HPW // L, pos_body(posvb, HPW // L), rs1)

        @pl.when(c == 0)
        def _wpos():
            pltpu.sync_copy(posva, pos_hbm.at[pl.ds(sbase, HPW)])
            pltpu.sync_copy(posvb, pos_hbm.at[pl.ds(sbase + HPW, HPW)])

        @pl.when((c == 0) & (s == 0))
        def _bex():
            b1 = jnp.zeros((L,), jnp.int32)
            b2 = jnp.zeros((L,), jnp.int32)
            for e in range(E):
                sev = splat(segend, e)
                b1 = b1 + jnp.where(lanes * TB >= sev, 1, 0).astype(jnp.int32)
                b2 = b2 + jnp.where((lanes + L) * TB >= sev,
                                    1, 0).astype(jnp.int32)
            bexbuf[pl.ds(0, L)] = jnp.minimum(b1, E - 1)
            bexbuf[pl.ds(L, L)] = jnp.minimum(b2, E - 1)
            pltpu.sync_copy(bexbuf, bex_hbm)

        @pl.when(c == 0)
        def _sc0():
            pltpu.sync_copy(x_hbm.at[pl.ds(sbase, HPW)], rows)
            pltpu.async_copy(rows, xs_hbm.at[posva], sem).wait()

        @pl.when(c == 1)
        def _sc1():
            pltpu.sync_copy(x_hbm.at[pl.ds(sbase + HPW, HPW)], rows)
            pltpu.async_copy(rows, xs_hbm.at[posvb], sem).wait()

    return route_gather


def _make_out_gather(T, OUT, P):
    NC, NS = 2, 16
    NW = NC * NS
    RPW = T // NW
    HC = RPW // 4
    mesh = plsc.VectorSubcoreMesh(core_axis_name="c", subcore_axis_name="s")

    @functools.partial(
        pl.kernel, mesh=mesh,
        out_type=jax.ShapeDtypeStruct((T, OUT), jnp.float32),
        scratch_types=[
            pltpu.VMEM((HC,), jnp.int32),
            pltpu.VMEM((HC,), jnp.int32),
            pltpu.VMEM((HC, OUT), jnp.float32),
            pltpu.VMEM((HC, OUT), jnp.float32),
            pltpu.SemaphoreType.DMA,
            pltpu.SemaphoreType.DMA,
        ],
    )
    def out_gather(outs_hbm, pos_hbm, out_hbm, posa, posb, rowsa, rowsb,
                   sema, semb):
        c = lax.axis_index("c")
        s = lax.axis_index("s")
        wid = s * NC + c
        base = pl.multiple_of(RPW * wid, RPW)
        pltpu.sync_copy(pos_hbm.at[pl.ds(base, HC)], posa)
        cps = [None, None]
        bufs = [rowsa, rowsb]
        sems = [sema, semb]
        cps[0] = pltpu.async_copy(outs_hbm.at[posa], rowsa, sema)
        for h in range(1, 4):
            pbuf = posa if (h % 2 == 0) else posb
            pltpu.sync_copy(pos_hbm.at[pl.ds(base + h * HC, HC)], pbuf)
            cps[h % 2] = pltpu.async_copy(outs_hbm.at[pbuf], bufs[h % 2],
                                          sems[h % 2])
            cps[(h - 1) % 2].wait()
            pltpu.sync_copy(bufs[(h - 1) % 2],
                            out_hbm.at[pl.ds(base + (h - 1) * HC, HC)])
        cps[1].wait()
        pltpu.sync_copy(bufs[1], out_hbm.at[pl.ds(base + 3 * HC, HC)])

    return out_gather


def _gmm_body(bex_ref, xs_ref, w_ref, a_ref, b_ref, dw_ref, metas_ref, ss_ref,
              out_ref):
    xs32 = xs_ref[...]
    xb = xs32.astype(jnp.bfloat16)
    base = jax.lax.dot_general(
        xb, w_ref[...].astype(jnp.bfloat16), (((1,), (1,)), ((), ())),
        preferred_element_type=jnp.float32)
    mid = jnp.dot(xb, a_ref[0].astype(jnp.bfloat16),
                  preferred_element_type=jnp.float32)
    b = b_ref[0].astype(jnp.bfloat16)
    R, OUT = b.shape
    col = jax.lax.broadcasted_iota(jnp.int32, (R, OUT), 1)
    zero = jnp.zeros_like(b)
    row0 = jnp.where(col < OUT // 2, b, zero)
    row1 = jnp.where(col >= OUT // 2, b, zero)
    bfull = jnp.concatenate([row0, row1], axis=0)
    lora = jnp.dot(mid.astype(jnp.bfloat16), bfull,
                   preferred_element_type=jnp.float32)
    dmm = jnp.dot(xb, dw_ref[0].astype(jnp.bfloat16),
                  preferred_element_type=jnp.float32)
    rs = jnp.sum(xs32, axis=1, keepdims=True)
    out_ref[...] = base + lora + dmm * ss_ref[0] + rs * metas_ref[0]


def _grouped_matmul(bex, xs, W, A, B, DW, metas, ss, P):
    E, D, R2 = A.shape
    R = R2 // 2
    OUT = B.shape[2]
    ntb = P // TB

    def emap(tb, bex_ref):
        return (bex_ref[tb], 0, 0)

    grid_spec = pltpu.PrefetchScalarGridSpec(
        num_scalar_prefetch=1,
        grid=(ntb,),
        in_specs=[
            pl.BlockSpec((TB, D), lambda tb, bex_ref: (tb, 0)),
            pl.BlockSpec((OUT, D), lambda tb, bex_ref: (0, 0)),
            pl.BlockSpec((1, D, R2), emap),
            pl.BlockSpec((1, R, OUT), emap),
            pl.BlockSpec((1, D, OUT), emap),
            pl.BlockSpec((1, 1, OUT), emap),
            pl.BlockSpec((1, 1, OUT), emap),
        ],
        out_specs=pl.BlockSpec((TB, OUT), lambda tb, bex_ref: (tb, 0)),
    )
    return pl.pallas_call(
        _gmm_body,
        grid_spec=grid_spec,
        out_shape=jax.ShapeDtypeStruct((P, OUT), jnp.float32),
        compiler_params=pltpu.CompilerParams(
            dimension_semantics=("arbitrary",)),
    )(bex, xs, W, A, B, DW, metas, ss)


def kernel(input_, weight_indices, W, A_buffer, B_buffer, DeltaW_buffer,
           metas_buffer, ss_buffer):
    T, D = input_.shape
    E = A_buffer.shape[0]
    OUT = B_buffer.shape[2]
    P = T + E * TB
    idx = weight_indices.astype(jnp.int32)

    xs, pos, bex = _make_route_gather(T, D, E, P)(idx, input_)

    out_sorted = _grouped_matmul(
        bex, xs, W, A_buffer, B_buffer, DeltaW_buffer,
        metas_buffer, ss_buffer, P)

    return _make_out_gather(T, OUT, P)(out_sorted, pos)


# speedup vs baseline: 1.9276x; 1.0100x over previous
import functools

import jax
import jax.numpy as jnp
from jax import lax
from jax.experimental import pallas as pl
from jax.experimental.pallas import tpu as pltpu
from jax.experimental.pallas import tpu_sc as plsc

TB = 128
CB = 1024
L = 16


def _make_route_gather(T, D, E, P):
    NC, NS = 2, 16
    TPW = T // NS
    HPW = TPW // 2
    mesh = plsc.VectorSubcoreMesh(core_axis_name="c", subcore_axis_name="s")

    @functools.partial(
        pl.kernel, mesh=mesh,
        out_type=[
            jax.ShapeDtypeStruct((P, D), jnp.float32),
            jax.ShapeDtypeStruct((T,), jnp.int32),
            jax.ShapeDtypeStruct((2 * L,), jnp.int32),
        ],
        scratch_types=[
            pltpu.VMEM((TPW,), jnp.int32),
            pltpu.VMEM((HPW,), jnp.int32),
            pltpu.VMEM((HPW,), jnp.int32),
            pltpu.VMEM((HPW, D), jnp.float32),
            pltpu.VMEM((L,), jnp.int32),
            pltpu.VMEM((NS * L,), jnp.int32),
            pltpu.VMEM((2 * L,), jnp.int32),
            pltpu.VMEM_SHARED((NS * L,), jnp.int32),
            pltpu.SemaphoreType.DMA,
        ],
    )
    def route_gather(idx_hbm, x_hbm, xs_hbm, pos_hbm, bex_hbm,
                     idx_v, posva, posvb, rows, cntbuf, cmatv, bexbuf,
                     cnt_sh, sem):
        c = lax.axis_index("c")
        s = lax.axis_index("s")
        lanes = lax.iota(jnp.int32, L)
        sbase = pl.multiple_of(s * TPW, TPW)
        pltpu.sync_copy(idx_hbm.at[pl.ds(sbase, TPW)], idx_v)

        def lane_gather(v, idx):
            return lax.gather(
                v, idx[:, None],
                dimension_numbers=lax.GatherDimensionNumbers(
                    offset_dims=(), collapsed_slice_dims=(0,),
                    start_index_map=(0,)),
                slice_sizes=(1,),
                mode=lax.GatherScatterMode.PROMISE_IN_BOUNDS)

        def splat(v, k):
            return lane_gather(v, jnp.full((L,), k, jnp.int32))

        def prefix_incl(v):
            for k in (1, 2, 4, 8):
                shifted = lane_gather(v, jnp.maximum(lanes - k, 0))
                v = v + jnp.where(lanes >= k, shifted, 0)
            return v

        def chunk_ranks(v, rstart):
            vals = jnp.zeros((L,), jnp.int32)
            add = jnp.zeros((L,), jnp.int32)
            for e in range(E):
                m = v == e
                mi = jnp.where(m, jnp.int32(1), jnp.int32(0))
                pres = prefix_incl(mi)
                tot = splat(pres, L - 1)
                vals = jnp.where(m, splat(rstart, e) + pres - 1, vals)
                add = add + jnp.where(lanes == e, tot, 0)
            return vals, rstart + add

        def cnt_body(i, rs):
            v = idx_v[pl.ds(i * L, L)]
            _, rs = chunk_ranks(v, rs)
            return rs

        mycnt = lax.fori_loop(0, TPW // L, cnt_body,
                              jnp.zeros((L,), jnp.int32))
        cntbuf[...] = mycnt
        pltpu.sync_copy(cntbuf,
                        cnt_sh.at[pl.ds(pl.multiple_of(s * L, L), L)])
        plsc.subcore_barrier()
        pltpu.sync_copy(cnt_sh, cmatv)

        total = jnp.zeros((L,), jnp.int32)
        mypre = jnp.zeros((L,), jnp.int32)
        for s2 in range(NS):
            row = cmatv[pl.ds(s2 * L, L)]
            total = total + row
            mypre = mypre + jnp.where(s2 < s, row, 0)
        seg = jnp.bitwise_and(total + (TB - 1), -TB)
        segend = prefix_incl(seg)
        off = segend - seg
        rstart0 = off + mypre

        def pos_body(tref, base_ci):
            def body(i, rs):
                v = idx_v[pl.ds(pl.multiple_of((base_ci + i) * L, L), L)]
                vals, rs = chunk_ranks(v, rs)
                tref[pl.ds(i * L, L)] = vals
                return rs

            return body

        rs1 = lax.fori_loop(0, HPW // L, pos_body(posva, 0), rstart0)
        lax.fori_loop(0, HPW // L, pos_body(posvb, HPW // L), rs1)

        @pl.when(c == 0)
        def _wpos():
            pltpu.sync_copy(posva, pos_hbm.at[pl.ds(sbase, HPW)])
            pltpu.sync_copy(posvb, pos_hbm.at[pl.ds(sbase + HPW, HPW)])

        @pl.when((c == 0) & (s == 0))
        def _bex():
            b1 = jnp.zeros((L,), jnp.int32)
            b2 = jnp.zeros((L,), jnp.int32)
            for e in range(E):
                sev = splat(segend, e)
                b1 = b1 + jnp.where(lanes * TB >= sev, 1, 0).astype(jnp.int32)
                b2 = b2 + jnp.where((lanes + L) * TB >= sev,
                                    1, 0).astype(jnp.int32)
            bexbuf[pl.ds(0, L)] = jnp.minimum(b1, E - 1)
            bexbuf[pl.ds(L, L)] = jnp.minimum(b2, E - 1)
            pltpu.sync_copy(bexbuf, bex_hbm)

        @pl.when(c == 0)
        def _sc0():
            pltpu.sync_copy(x_hbm.at[pl.ds(sbase, HPW)], rows)
            pltpu.async_copy(rows, xs_hbm.at[posva], sem).wait()

        @pl.when(c == 1)
        def _sc1():
            pltpu.sync_copy(x_hbm.at[pl.ds(sbase + HPW, HPW)], rows)
            pltpu.async_copy(rows, xs_hbm.at[posvb], sem).wait()

    return route_gather


def _make_out_gather(T, OUT, P):
    NC, NS = 2, 16
    NW = NC * NS
    RPW = T // NW
    HC = RPW // 4
    mesh = plsc.VectorSubcoreMesh(core_axis_name="c", subcore_axis_name="s")

    @functools.partial(
        pl.kernel, mesh=mesh,
        out_type=jax.ShapeDtypeStruct((T, OUT), jnp.float32),
        scratch_types=[
            pltpu.VMEM((HC,), jnp.int32),
            pltpu.VMEM((HC,), jnp.int32),
            pltpu.VMEM((HC, OUT), jnp.float32),
            pltpu.VMEM((HC, OUT), jnp.float32),
            pltpu.SemaphoreType.DMA,
            pltpu.SemaphoreType.DMA,
        ],
    )
    def out_gather(outs_hbm, pos_hbm, out_hbm, posa, posb, rowsa, rowsb,
                   sema, semb):
        c = lax.axis_index("c")
        s = lax.axis_index("s")
        wid = s * NC + c
        base = pl.multiple_of(RPW * wid, RPW)
        pltpu.sync_copy(pos_hbm.at[pl.ds(base, HC)], posa)
        cps = [None, None]
        bufs = [rowsa, rowsb]
        sems = [sema, semb]
        cps[0] = pltpu.async_copy(outs_hbm.at[posa], rowsa, sema)
        for h in range(1, 4):
            pbuf = posa if (h % 2 == 0) else posb
            pltpu.sync_copy(pos_hbm.at[pl.ds(base + h * HC, HC)], pbuf)
            cps[h % 2] = pltpu.async_copy(outs_hbm.at[pbuf], bufs[h % 2],
                                          sems[h % 2])
            cps[(h - 1) % 2].wait()
            pltpu.sync_copy(bufs[(h - 1) % 2],
                            out_hbm.at[pl.ds(base + (h - 1) * HC, HC)])
        cps[1].wait()
        pltpu.sync_copy(bufs[1], out_hbm.at[pl.ds(base + 3 * HC, HC)])

    return out_gather


def _gmm_body(bex_ref, xs_ref, w_ref, a_ref, b_ref, dw_ref, metas_ref, ss_ref,
              out_ref):
    xs32 = xs_ref[...]
    xb = xs32.astype(jnp.bfloat16)
    base = jax.lax.dot_general(
        xb, w_ref[...].astype(jnp.bfloat16), (((1,), (1,)), ((), ())),
        preferred_element_type=jnp.float32)
    mid = jnp.dot(xb, a_ref[0],
                  preferred_element_type=jnp.float32)
    lora = jnp.dot(mid.astype(jnp.bfloat16), b_ref[0],
                   preferred_element_type=jnp.float32)
    dmm = jnp.dot(xb, dw_ref[0].astype(jnp.bfloat16),
                  preferred_element_type=jnp.float32)
    rs = jnp.sum(xs32, axis=1, keepdims=True)
    out_ref[...] = base + lora + dmm * ss_ref[0] + rs * metas_ref[0]


def _grouped_matmul(bex, xs, W, A, B, DW, metas, ss, P):
    E, D, R2 = A.shape
    OUT = B.shape[2]
    ntb = P // TB

    def emap(tb, bex_ref):
        return (bex_ref[tb], 0, 0)

    grid_spec = pltpu.PrefetchScalarGridSpec(
        num_scalar_prefetch=1,
        grid=(ntb,),
        in_specs=[
            pl.BlockSpec((TB, D), lambda tb, bex_ref: (tb, 0)),
            pl.BlockSpec((OUT, D), lambda tb, bex_ref: (0, 0)),
            pl.BlockSpec((1, D, R2), emap),
            pl.BlockSpec((1, R2, OUT), emap),
            pl.BlockSpec((1, D, OUT), emap),
            pl.BlockSpec((1, 1, OUT), emap),
            pl.BlockSpec((1, 1, OUT), emap),
        ],
        out_specs=pl.BlockSpec((TB, OUT), lambda tb, bex_ref: (tb, 0)),
    )
    return pl.pallas_call(
        _gmm_body,
        grid_spec=grid_spec,
        out_shape=jax.ShapeDtypeStruct((P, OUT), jnp.float32),
        compiler_params=pltpu.CompilerParams(
            dimension_semantics=("arbitrary",)),
    )(bex, xs, W, A, B, DW, metas, ss)


def kernel(input_, weight_indices, W, A_buffer, B_buffer, DeltaW_buffer,
           metas_buffer, ss_buffer):
    T, D = input_.shape
    E = A_buffer.shape[0]
    OUT = B_buffer.shape[2]
    P = T + E * TB
    idx = weight_indices.astype(jnp.int32)

    R = A_buffer.shape[2] // 2
    bd = B_buffer.shape[2] // 2
    Bp = jnp.zeros((E, 2 * R, 2 * bd), jnp.bfloat16)
    Bp = Bp.at[:, :R, :bd].set(B_buffer[:, :, :bd].astype(jnp.bfloat16))
    Bp = Bp.at[:, R:, bd:].set(B_buffer[:, :, bd:].astype(jnp.bfloat16))
    Ab = A_buffer.astype(jnp.bfloat16)

    xs, pos, bex = _make_route_gather(T, D, E, P)(idx, input_)

    out_sorted = _grouped_matmul(
        bex, xs, W, Ab, Bp, DeltaW_buffer,
        metas_buffer, ss_buffer, P)

    return _make_out_gather(T, OUT, P)(out_sorted, pos)
